# Initial kernel scaffold; baseline (speedup 1.0000x reference)
import jax
import jax.numpy as jnp
from jax.experimental import pallas as pl


def kernel(fmap0, fmap1, fmap2, w0, w1, w2, b0, b1, b2):
    raise NotImplementedError("write your pallas kernel here")


# speedup vs baseline: 1.1252x; 1.1252x over previous
import jax
import jax.numpy as jnp
import numpy as np
from jax import lax
from jax.experimental import pallas as pl
from jax.experimental.pallas import tpu as pltpu


def _pool_1d(n_in, n_out):
    p = np.zeros((n_in, n_out), np.float32)
    for i in range(n_out):
        s = (i * n_in) // n_out
        e = -(-((i + 1) * n_in) // n_out)
        p[s:e, i] = 1.0 / (e - s)
    return p


def _pool_matrix(in_hw, out_hw):
    hi, wi = in_hw
    ho, wo = out_hw
    ph = _pool_1d(hi, ho)
    pw = _pool_1d(wi, wo)
    return np.einsum("ih,jw->ijhw", ph, pw).reshape(hi * wi, ho * wo)


def _layer(x_ref, w_ref, brow_ref, bcol_ref, emb_ref, cert_ref):
    x = x_ref[0]
    hw = x.shape[1]
    inv_hw = jnp.float32(1.0 / hw)
    inv_hw_m1 = jnp.float32(1.0 / max(hw - 1, 1))

    mx = jnp.max(x, axis=-1, keepdims=True)
    mean = jnp.sum(x, axis=-1, keepdims=True) * inv_hw
    pooled = (mean + mx).astype(jnp.bfloat16)

    w = w_ref[...]
    emb = lax.dot_general(pooled, w, (((0,), (1,)), ((), ())),
                          preferred_element_type=jnp.float32)
    emb_ref[0] = emb + brow_ref[...]

    xp = jnp.where(x == mx, x * jnp.float32(hw + 1), x).astype(jnp.bfloat16)
    cam = lax.dot_general(w, xp, (((1,), (0,)), ((), ())),
                          preferred_element_type=jnp.float32) + bcol_ref[...]

    m = jnp.sum(cam, axis=-1, keepdims=True) * inv_hw
    d = cam - m
    var = jnp.sum(d * d, axis=-1) * inv_hw_m1
    cert_ref[0, 0] = jnp.sqrt(var)
    return cam


def _pooled_low(cam, p_ref):
    lp = lax.dot_general(cam.astype(jnp.bfloat16), p_ref[...],
                         (((1,), (0,)), ((), ())),
                         preferred_element_type=jnp.float32)
    inv = lax.rsqrt(jnp.maximum(
        jnp.sum(lp * lp, axis=-1, keepdims=True), 1e-24))
    return (lp * inv).astype(jnp.bfloat16)


def _accum_link(link_ref, low_n, cam_hi, inv_batch):
    inv_h = lax.rsqrt(jnp.maximum(
        jnp.sum(cam_hi * cam_hi, axis=-1, keepdims=True), 1e-24))
    hi_n = (cam_hi * inv_h).astype(jnp.bfloat16)
    g = lax.dot_general(low_n, hi_n, (((1,), (1,)), ((), ())),
                        preferred_element_type=jnp.float32)

    @pl.when(pl.program_id(1) == 0)
    def _():
        link_ref[...] = jnp.zeros_like(link_ref)

    link_ref[0] += g * jnp.float32(inv_batch)


def _fused_kernel(x0_ref, x1_ref, x2_ref, w0_ref, w1_ref, w2_ref,
                  br0_ref, br1_ref, br2_ref, bc0_ref, bc1_ref, bc2_ref,
                  p0_ref, p1_ref,
                  emb0_ref, emb1_ref, emb2_ref,
                  cert0_ref, cert1_ref, cert2_ref,
                  l0_ref, l1_ref, *, inv_batch):
    cam0 = _layer(x0_ref, w0_ref, br0_ref, bc0_ref, emb0_ref, cert0_ref)
    low0 = _pooled_low(cam0, p0_ref)

    cam1 = _layer(x1_ref, w1_ref, br1_ref, bc1_ref, emb1_ref, cert1_ref)
    _accum_link(l0_ref, low0, cam1, inv_batch)
    low1 = _pooled_low(cam1, p1_ref)

    cam2 = _layer(x2_ref, w2_ref, br2_ref, bc2_ref, emb2_ref, cert2_ref)
    _accum_link(l1_ref, low1, cam2, inv_batch)


def kernel(fmap0, fmap1, fmap2, w0, w1, w2, b0, b1, b2):
    import functools

    fmaps = [fmap0, fmap1, fmap2]
    B = fmap0.shape[0]
    R = w0.shape[0]
    Cs = [f.shape[1] for f in fmaps]
    spatial = [(f.shape[2], f.shape[3]) for f in fmaps]
    HWs = [h * w for (h, w) in spatial]
    xs = [f.astype(jnp.float32).reshape(B, c, hw)
          for f, c, hw in zip(fmaps, Cs, HWs)]

    ws = [w.astype(jnp.bfloat16) for w in (w0, w1, w2)]
    brows = [b.reshape(1, R) for b in (b0, b1, b2)]
    bcols = [b.reshape(R, 1) for b in (b0, b1, b2)]
    p0 = jnp.asarray(_pool_matrix(spatial[0], spatial[1]), jnp.bfloat16)
    p1 = jnp.asarray(_pool_matrix(spatial[1], spatial[2]), jnp.bfloat16)

    NC = 2 if B % 2 == 0 else 1
    JB = B // NC

    def x_spec(c, hw):
        return pl.BlockSpec((1, c, hw), lambda i, j, JB=JB: (i * JB + j, 0, 0))

    def const_spec(shape):
        return pl.BlockSpec(shape, lambda i, j: (0,) * len(shape))

    def out_spec():
        return pl.BlockSpec((1, 1, R), lambda i, j, JB=JB: (i * JB + j, 0, 0))

    br_shape = jax.ShapeDtypeStruct((B, 1, R), jnp.float32)
    outs = pl.pallas_call(
        functools.partial(_fused_kernel, inv_batch=1.0 / B),
        grid=(NC, JB),
        in_specs=[
            x_spec(Cs[0], HWs[0]), x_spec(Cs[1], HWs[1]), x_spec(Cs[2], HWs[2]),
            const_spec((R, Cs[0])), const_spec((R, Cs[1])), const_spec((R, Cs[2])),
            const_spec((1, R)), const_spec((1, R)), const_spec((1, R)),
            const_spec((R, 1)), const_spec((R, 1)), const_spec((R, 1)),
            const_spec((HWs[0], HWs[1])), const_spec((HWs[1], HWs[2])),
        ],
        out_specs=[
            out_spec(), out_spec(), out_spec(),
            out_spec(), out_spec(), out_spec(),
            pl.BlockSpec((1, R, R), lambda i, j: (i, 0, 0)),
            pl.BlockSpec((1, R, R), lambda i, j: (i, 0, 0)),
        ],
        out_shape=[
            br_shape, br_shape, br_shape,
            br_shape, br_shape, br_shape,
            jax.ShapeDtypeStruct((NC, R, R), jnp.float32),
            jax.ShapeDtypeStruct((NC, R, R), jnp.float32),
        ],
        compiler_params=pltpu.CompilerParams(
            dimension_semantics=("parallel", "arbitrary"),
            vmem_limit_bytes=64 * 1024 * 1024,
        ),
    )(xs[0], xs[1], xs[2], ws[0], ws[1], ws[2],
      brows[0], brows[1], brows[2], bcols[0], bcols[1], bcols[2], p0, p1)

    emb0, emb1, emb2, cert0, cert1, cert2, l0, l1 = outs
    embeddings = [e.reshape(B, R) for e in (emb0, emb1, emb2)]
    certainties = [c.reshape(B, R) for c in (cert0, cert1, cert2)]
    links = [jnp.sum(l0, axis=0), jnp.sum(l1, axis=0)]
    return embeddings, certainties, links
